# baseline (speedup 1.0000x reference)
import functools

import jax
import jax.numpy as jnp
from jax.experimental import pallas as pl
from jax.experimental.pallas import tpu as pltpu

_K = 16
_ROW_BLK = 256
_NORM_BLK = 512


def _norm_kernel(x_ref, xn_ref, sq_ref):
    x = x_ref[0]
    ssq = jnp.sum(x * x, axis=1, keepdims=True)
    xn = x / jnp.maximum(jnp.sqrt(ssq), 1e-12)
    xn_ref[0] = xn.astype(jnp.bfloat16)
    sq_ref[0] = jax.lax.dot_general(
        jnp.ones((1, x.shape[1]), jnp.float32), xn * xn,
        (((1,), (1,)), ((), ())), preferred_element_type=jnp.float32,
        precision=jax.lax.Precision.HIGHEST)


_NEG = -1e30
_LVL = 4


def _full_scan(a16, bm16, sqv, r, n, k):
    inner = jax.lax.dot_general(a16, bm16, (((1,), (1,)), ((), ())),
                                preferred_element_type=jnp.float32)
    work = 2.0 * inner - sqv
    iota = jax.lax.broadcasted_iota(jnp.int32, (r, n), 1)
    lane = jax.lax.broadcasted_iota(jnp.int32, (r, k), 1)
    nn = jnp.zeros((r, k), jnp.int32)
    for t in range(k + 1):
        m = jnp.max(work, axis=1, keepdims=True)
        cand = jnp.where(work == m, iota, n)
        idx = jnp.min(cand, axis=1, keepdims=True)
        if t > 0:
            nn = jnp.where(lane == t - 1, idx, nn)
        if t < k:
            work = jnp.where(iota == idx, -1e30, work)
    return nn


def _topk_kernel(xr_ref, xa_ref, sq_ref, nn_ref, ct_ref, *, n, k, r):
    b = pl.program_id(0)
    i = pl.program_id(1)
    a16 = xr_ref[0]
    bm16 = xa_ref[0]

    lane128 = jax.lax.broadcasted_iota(jnp.int32, (r, 128), 1)
    gs = [jnp.full((r, 128), _NEG, jnp.float32) for _ in range(_LVL)]
    cs = [jnp.full((r, 128), n, jnp.int32) for _ in range(_LVL)]
    for v in range(n // 128):
        sc = jax.lax.dot_general(a16, bm16[v * 128:(v + 1) * 128, :],
                                 (((1,), (1,)), ((), ())),
                                 preferred_element_type=jnp.float32)
        val = 2.0 * sc - sq_ref[0][:, v * 128:(v + 1) * 128]
        col = lane128 + v * 128
        bs = [val > g for g in gs]
        ng = [jnp.where(bs[0], val, gs[0])]
        nc = [jnp.where(bs[0], col, cs[0])]
        for q in range(1, _LVL):
            ng.append(jnp.where(bs[q], jnp.where(bs[q - 1], gs[q - 1], val),
                                gs[q]))
            nc.append(jnp.where(bs[q], jnp.where(bs[q - 1], cs[q - 1], col),
                                cs[q]))
        gs, cs = ng, nc

    lane = jax.lax.broadcasted_iota(jnp.int32, (r, k), 1)
    nn = jnp.zeros((r, k), jnp.int32)
    d = jnp.zeros((r, 128), jnp.int32)
    badp = jnp.zeros((r, 128), jnp.bool_)
    for t in range(k + 1):
        m = jnp.max(gs[0], axis=1, keepdims=True)
        j = jnp.min(jnp.where(gs[0] == m, cs[0], n), axis=1, keepdims=True)
        if t > 0:
            nn = jnp.where(lane == t - 1, j, nn)
        if t < k:
            gm = lane128 == jnp.bitwise_and(j, 127)
            badp = badp | (gm & (d >= _LVL - 1))
            d = d + gm.astype(jnp.int32)
            for q in range(_LVL - 1):
                gs[q] = jnp.where(gm, gs[q + 1], gs[q])
                cs[q] = jnp.where(gm, cs[q + 1], cs[q])
            gs[_LVL - 1] = jnp.where(gm, _NEG, gs[_LVL - 1])
            cs[_LVL - 1] = jnp.where(gm, n, cs[_LVL - 1])
    bad = jnp.any(badp)
    nn = jax.lax.cond(
        bad, lambda: _full_scan(a16, bm16, sq_ref[0], r, n, k), lambda: nn)

    off = b * n
    nn_ref[0] = nn + off
    ct_ref[0] = jax.lax.broadcasted_iota(jnp.int32, (r, k), 0) + (off + i * r)


def kernel(x):
    B, N, C = x.shape
    rn = min(_NORM_BLK, N)
    xn, sq = pl.pallas_call(
        _norm_kernel,
        grid=(B, N // rn),
        in_specs=[pl.BlockSpec((1, rn, C), lambda b, i: (b, i, 0))],
        out_specs=[
            pl.BlockSpec((1, rn, C), lambda b, i: (b, i, 0)),
            pl.BlockSpec((1, 1, rn), lambda b, i: (b, 0, i)),
        ],
        out_shape=[
            jax.ShapeDtypeStruct((B, N, C), jnp.bfloat16),
            jax.ShapeDtypeStruct((B, 1, N), jnp.float32),
        ],
        compiler_params=pltpu.CompilerParams(
            dimension_semantics=("parallel", "parallel")),
    )(x)

    r = min(_ROW_BLK, N)
    nn, ct = pl.pallas_call(
        functools.partial(_topk_kernel, n=N, k=_K, r=r),
        grid=(B, N // r),
        in_specs=[
            pl.BlockSpec((1, r, C), lambda b, i: (b, i, 0)),
            pl.BlockSpec((1, N, C), lambda b, i: (b, 0, 0)),
            pl.BlockSpec((1, 1, N), lambda b, i: (b, 0, 0)),
        ],
        out_specs=[
            pl.BlockSpec((1, r, _K), lambda b, i: (b, i, 0)),
            pl.BlockSpec((1, r, _K), lambda b, i: (b, i, 0)),
        ],
        out_shape=[
            jax.ShapeDtypeStruct((B, N, _K), jnp.int32),
            jax.ShapeDtypeStruct((B, N, _K), jnp.int32),
        ],
        compiler_params=pltpu.CompilerParams(
            dimension_semantics=("parallel", "arbitrary")),
    )(xn, xn, sq)

    edge_index = jnp.stack((nn.reshape(-1), ct.reshape(-1)), axis=0)
    return edge_index.astype(jnp.int64)


# speedup vs baseline: 2.1362x; 2.1362x over previous
import functools

import jax
import jax.numpy as jnp
from jax.experimental import pallas as pl
from jax.experimental.pallas import tpu as pltpu

_K = 16
_ROW_BLK = 256
_NORM_BLK = 512


def _norm_kernel(x_ref, xn_ref, sq_ref):
    x = x_ref[0]
    ssq = jnp.sum(x * x, axis=1, keepdims=True)
    xn = x / jnp.maximum(jnp.sqrt(ssq), 1e-12)
    xn_ref[0] = xn.astype(jnp.bfloat16)
    sq_ref[0] = jax.lax.dot_general(
        jnp.ones((1, x.shape[1]), jnp.float32), xn * xn,
        (((1,), (1,)), ((), ())), preferred_element_type=jnp.float32,
        precision=jax.lax.Precision.HIGHEST)


_NEG = -1e30
_LVL = 5


def _full_scan(a16, bm16, sqv, r, n, k):
    inner = jax.lax.dot_general(a16, bm16, (((1,), (1,)), ((), ())),
                                preferred_element_type=jnp.float32)
    work = 2.0 * inner - sqv
    iota = jax.lax.broadcasted_iota(jnp.int32, (r, n), 1)
    lane = jax.lax.broadcasted_iota(jnp.int32, (r, k), 1)
    nn = jnp.zeros((r, k), jnp.int32)
    for t in range(k + 1):
        m = jnp.max(work, axis=1, keepdims=True)
        cand = jnp.where(work == m, iota, n)
        idx = jnp.min(cand, axis=1, keepdims=True)
        if t > 0:
            nn = jnp.where(lane == t - 1, idx, nn)
        if t < k:
            work = jnp.where(iota == idx, -1e30, work)
    return nn


def _topk_kernel(xr_ref, xa_ref, sq_ref, nn_ref, ct_ref, *, n, k, r):
    b = pl.program_id(0)
    i = pl.program_id(1)
    a16 = xr_ref[0]
    bm16 = xa_ref[0]

    lane128 = jax.lax.broadcasted_iota(jnp.int32, (r, 128), 1)
    gs = [jnp.full((r, 128), _NEG, jnp.float32) for _ in range(_LVL)]
    cs = [jnp.full((r, 128), n, jnp.int32) for _ in range(_LVL)]
    for v in range(n // 128):
        sc = jax.lax.dot_general(a16, bm16[v * 128:(v + 1) * 128, :],
                                 (((1,), (1,)), ((), ())),
                                 preferred_element_type=jnp.float32)
        val = 2.0 * sc - sq_ref[0][:, v * 128:(v + 1) * 128]
        col = lane128 + v * 128
        bs = [val > g for g in gs]
        ng = [jnp.where(bs[0], val, gs[0])]
        nc = [jnp.where(bs[0], col, cs[0])]
        for q in range(1, _LVL):
            ng.append(jnp.where(bs[q], jnp.where(bs[q - 1], gs[q - 1], val),
                                gs[q]))
            nc.append(jnp.where(bs[q], jnp.where(bs[q - 1], cs[q - 1], col),
                                cs[q]))
        gs, cs = ng, nc

    lane = jax.lax.broadcasted_iota(jnp.int32, (r, k), 1)
    nn = jnp.zeros((r, k), jnp.int32)
    d = jnp.zeros((r, 128), jnp.int32)
    badp = jnp.zeros((r, 128), jnp.bool_)
    for t in range(k + 1):
        m = jnp.max(gs[0], axis=1, keepdims=True)
        j = jnp.min(jnp.where(gs[0] == m, cs[0], n), axis=1, keepdims=True)
        if t > 0:
            nn = jnp.where(lane == t - 1, j, nn)
        if t < k:
            gm = lane128 == jnp.bitwise_and(j, 127)
            badp = badp | (gm & (d >= _LVL - 1))
            d = d + gm.astype(jnp.int32)
            for q in range(_LVL - 1):
                gs[q] = jnp.where(gm, gs[q + 1], gs[q])
                cs[q] = jnp.where(gm, cs[q + 1], cs[q])
            gs[_LVL - 1] = jnp.where(gm, _NEG, gs[_LVL - 1])
            cs[_LVL - 1] = jnp.where(gm, n, cs[_LVL - 1])
    bad = jnp.any(badp)
    nn = jax.lax.cond(
        bad, lambda: _full_scan(a16, bm16, sq_ref[0], r, n, k), lambda: nn)

    off = b * n
    nn_ref[0] = nn + off
    ct_ref[0] = jax.lax.broadcasted_iota(jnp.int32, (r, k), 0) + (off + i * r)


def kernel(x):
    B, N, C = x.shape
    rn = min(_NORM_BLK, N)
    xn, sq = pl.pallas_call(
        _norm_kernel,
        grid=(B, N // rn),
        in_specs=[pl.BlockSpec((1, rn, C), lambda b, i: (b, i, 0))],
        out_specs=[
            pl.BlockSpec((1, rn, C), lambda b, i: (b, i, 0)),
            pl.BlockSpec((1, 1, rn), lambda b, i: (b, 0, i)),
        ],
        out_shape=[
            jax.ShapeDtypeStruct((B, N, C), jnp.bfloat16),
            jax.ShapeDtypeStruct((B, 1, N), jnp.float32),
        ],
        compiler_params=pltpu.CompilerParams(
            dimension_semantics=("parallel", "parallel")),
    )(x)

    r = min(_ROW_BLK, N)
    nn, ct = pl.pallas_call(
        functools.partial(_topk_kernel, n=N, k=_K, r=r),
        grid=(B, N // r),
        in_specs=[
            pl.BlockSpec((1, r, C), lambda b, i: (b, i, 0)),
            pl.BlockSpec((1, N, C), lambda b, i: (b, 0, 0)),
            pl.BlockSpec((1, 1, N), lambda b, i: (b, 0, 0)),
        ],
        out_specs=[
            pl.BlockSpec((1, r, _K), lambda b, i: (b, i, 0)),
            pl.BlockSpec((1, r, _K), lambda b, i: (b, i, 0)),
        ],
        out_shape=[
            jax.ShapeDtypeStruct((B, N, _K), jnp.int32),
            jax.ShapeDtypeStruct((B, N, _K), jnp.int32),
        ],
        compiler_params=pltpu.CompilerParams(
            dimension_semantics=("parallel", "arbitrary")),
    )(xn, xn, sq)

    edge_index = jnp.stack((nn.reshape(-1), ct.reshape(-1)), axis=0)
    return edge_index.astype(jnp.int64)
